# baseline (speedup 1.0000x reference)
import jax
import jax.numpy as jnp
from jax.experimental import pallas as pl
from jax.experimental.pallas import tpu as pltpu

_K = 5532
_C = 256
_KB = 1408
_NBLK = 4
_NPAD = _NBLK * _KB - _K
_NPIX = 2048
_N_TOT = 4096


def _oim_kernel(lut_ref, x_ref, tgt_ref, out_ref, s_ref, tl_ref):
    j = pl.program_id(0)
    b = pl.program_id(1)

    @pl.when(j == 0)
    def _init():
        s_ref[pl.ds(b, 1), :] = jnp.zeros((1, _NPIX), jnp.float32)
        tl_ref[pl.ds(b, 1), :] = jnp.zeros((1, _NPIX), jnp.float32)

    x = x_ref[b].astype(jnp.bfloat16)
    w = lut_ref[...].astype(jnp.bfloat16)
    wrow = jax.lax.broadcasted_iota(jnp.int32, (_KB, _C), 0) + j * _KB
    w = jnp.where(wrow < _K, w, jnp.bfloat16(0))
    s_blk = jax.lax.dot_general(w, x, (((1,), (0,)), ((), ())),
                                preferred_element_type=jnp.float32)

    s_ref[pl.ds(b, 1), :] += jnp.sum(jnp.exp(s_blk), axis=0, keepdims=True)

    row = jax.lax.broadcasted_iota(jnp.int32, (_KB, _NPIX), 0) + j * _KB
    eq = row == tgt_ref[b]
    tl_ref[pl.ds(b, 1), :] += jnp.sum(jnp.where(eq, s_blk, 0.0), axis=0,
                                      keepdims=True)

    @pl.when(j == _NBLK - 1)
    def _fin():
        s_row = s_ref[pl.ds(b, 1), :]
        tl_row = tl_ref[pl.ds(b, 1), :]
        nll = jnp.log(s_row - _NPAD) - tl_row
        part = jnp.sum(nll, axis=1, keepdims=True) * (1.0 / _N_TOT)

        @pl.when(b == 0)
        def _w():
            out_ref[...] = part

        @pl.when(b > 0)
        def _a():
            out_ref[...] += part


def kernel(lut, inputs, targets, epoch):
    x = inputs.reshape(2, _C, _NPIX)
    tgt = targets.reshape(2, 1, _NPIX)
    out = pl.pallas_call(
        _oim_kernel,
        grid=(_NBLK, 2),
        in_specs=[
            pl.BlockSpec((_KB, _C), lambda j, b: (j, 0)),
            pl.BlockSpec((2, _C, _NPIX), lambda j, b: (0, 0, 0)),
            pl.BlockSpec((2, 1, _NPIX), lambda j, b: (0, 0, 0)),
        ],
        out_specs=pl.BlockSpec((1, 1), lambda j, b: (0, 0)),
        out_shape=jax.ShapeDtypeStruct((1, 1), jnp.float32),
        scratch_shapes=[
            pltpu.VMEM((2, _NPIX), jnp.float32),
            pltpu.VMEM((2, _NPIX), jnp.float32),
        ],
        compiler_params=pltpu.CompilerParams(
            dimension_semantics=("arbitrary", "arbitrary"),
        ),
    )(lut, x, tgt)
    loss = out[0, 0]
    return jnp.where(epoch < 0, jnp.float32(0.0), loss)
